# baseline (speedup 1.0000x reference)
import functools

import jax
import jax.numpy as jnp
from jax import lax
from jax.experimental import pallas as pl
from jax.experimental.pallas import tpu as pltpu
from jax.experimental.pallas import tpu_sc as plsc

B = 4096
HIST = 20
NCAT = 4
EMB = 4
XW = 128
MW = 32

NC, NS, L = 2, 16, 16
NW = NC * NS
BPW = B // NW
NCHUNK = BPW // L

_REC0 = 0
_CAT0 = HIST
_UID, _GID, _JOB, _SEX, _AGE, _PUB = 24, 25, 26, 27, 28, 29

_OFF = {}
_cur = 0
for _name, _len in [
    ("fm1_uid", 944), ("fm1_gid", 1683), ("fm1_job", 22), ("fm1_sex", 2),
    ("fm1_age", 5), ("fm1_pub", 5),
    ("user", 944 * EMB), ("group", 1683 * EMB), ("cat", 20 * EMB),
    ("f2job", 22 * EMB), ("f2sex", 2 * EMB), ("f2age", 5 * EMB),
    ("f2pub", 5 * EMB),
]:
    _OFF[_name] = _cur
    _cur += _len
TAB_LEN = (_cur + 7) // 8 * 8

_TABLE_ARG_ORDER = ["fm1_uid", "fm1_gid", "fm1_job", "fm1_sex", "fm1_age",
                    "fm1_pub", "user", "group", "cat", "f2job", "f2sex",
                    "f2age", "f2pub"]


def _sc_body(meta_h, tab_h, xx_out, meta_v, tab_v, x_v, sem):
    wid = lax.axis_index("s") * NC + lax.axis_index("c")
    base = wid * BPW

    mrow = pl.multiple_of(base * MW // 128, 8)
    cps = [pltpu.make_async_copy(meta_h.at[pl.ds(mrow, BPW * MW // 128), :],
                                 meta_v, sem),
           pltpu.make_async_copy(tab_h, tab_v, sem)]
    for c in cps:
        c.start()
    for c in cps:
        c.wait()

    iota = jnp.arange(L, dtype=jnp.int32)

    def put(xb, col, val):
        plsc.store_scatter(x_v, [xb + col], val)

    def chunk(i):
        sidx = i * L + iota
        xb = lax.shift_left(sidx, 7)
        mr = lax.shift_right_logical(sidx, 2)
        mc = lax.shift_left(jnp.bitwise_and(sidx, 3), 5)

        def midx(c):
            return plsc.load_gather(meta_v, [mr, mc + c])

        uid = midx(_UID)
        gid = midx(_GID)
        job = midx(_JOB)
        sex = midx(_SEX)
        age = midx(_AGE)
        pub = midx(_PUB)

        def lg(idx):
            return plsc.load_gather(tab_v, [idx])

        fm1 = (lg(_OFF["fm1_uid"] + uid) + lg(_OFF["fm1_gid"] + gid)
               + lg(_OFF["fm1_job"] + job) + lg(_OFF["fm1_sex"] + sex)
               + lg(_OFF["fm1_age"] + age) + lg(_OFF["fm1_pub"] + pub))
        put(xb, 32, fm1)

        f_base = [
            _OFF["f2job"] + job * EMB,
            _OFF["f2sex"] + sex * EMB,
            _OFF["f2age"] + age * EMB,
            _OFF["f2pub"] + pub * EMB,
            _OFF["user"] + uid * EMB,
            _OFF["group"] + gid * EMB,
        ]
        rec_base = [_OFF["group"] + midx(_REC0 + h) * EMB
                    for h in range(HIST)]
        cat_base = [_OFF["cat"] + midx(_CAT0 + c) * EMB
                    for c in range(NCAT)]

        fm2_acc = jnp.zeros((L,), jnp.float32)
        for d in range(EMB):
            e = [lg(fb + d) for fb in f_base]
            hsum = lg(rec_base[0] + d)
            for rb in rec_base[1:]:
                hsum = hsum + lg(rb + d)
            csum = lg(cat_base[0] + d)
            for cb in cat_base[1:]:
                csum = csum + lg(cb + d)
            e.append(hsum)
            e.append(csum * 0.25)
            s = e[0]
            sq = e[0] * e[0]
            for ef in e[1:]:
                s = s + ef
                sq = sq + ef * ef
            fm2_acc = fm2_acc + (s * s - sq)
            for f in range(8):
                put(xb, f * EMB + d, e[f])
        put(xb, 33, fm2_acc * 0.5)

    for i in range(NCHUNK):
        chunk(jnp.int32(i))

    pltpu.sync_copy(x_v, xx_out.at[pl.ds(base * XW, BPW * XW)])


_sc_gather = functools.partial(
    pl.kernel,
    out_type=jax.ShapeDtypeStruct((B * XW,), jnp.float32),
    mesh=plsc.VectorSubcoreMesh(core_axis_name="c", subcore_axis_name="s"),
    scratch_types=[
        pltpu.VMEM((BPW * MW // 128, 128), jnp.int32),
        pltpu.VMEM((TAB_LEN,), jnp.float32),
        pltpu.VMEM((BPW * XW,), jnp.float32),
        pltpu.SemaphoreType.DMA,
    ],
    compiler_params=pltpu.CompilerParams(needs_layout_passes=False),
)(_sc_body)


def _dotT(a, b):
    return lax.dot_general(a, b, (((1,), (1,)), ((), ())),
                           preferred_element_type=jnp.float32)


def _tc_body(xx_ref, w1_ref, b1_ref, g1_ref, be1_ref,
             w2_ref, b2_ref, g2_ref, be2_ref, w3t_ref, b3_ref, g3_ref,
             be3_ref, wot_ref, bo_ref, out_ref):
    ones = jnp.full((1, B), 1.0 / B, jnp.float32)

    def bn_relu(h, b, g, be):
        n = h.shape[1]
        h = h + b.reshape(1, n)
        m = jnp.dot(ones, h, preferred_element_type=jnp.float32)
        ms = jnp.dot(ones, h * h, preferred_element_type=jnp.float32)
        v = ms - m * m
        scale = g.reshape(1, n) / jnp.sqrt(v + 1e-5)
        shift = be.reshape(1, n) - m * scale
        return jnp.maximum(h * scale + shift, 0.0)

    xx = jnp.reshape(xx_ref[...], (B, XW))
    lane = lax.broadcasted_iota(jnp.int32, (B, XW), 1)
    xm = jnp.where(lane < 34, xx, 0.0)
    w1p = jnp.concatenate(
        [w1_ref[...], jnp.zeros((XW - 32, 256), jnp.float32)], axis=0)
    h = bn_relu(jnp.dot(xm, w1p, preferred_element_type=jnp.float32),
                b1_ref[...], g1_ref[...], be1_ref[...])
    h = bn_relu(jnp.dot(h, w2_ref[...], preferred_element_type=jnp.float32),
                b2_ref[...], g2_ref[...], be2_ref[...])
    h = bn_relu(_dotT(h, w3t_ref[...]),
                b3_ref[...], g3_ref[...], be3_ref[...])
    wot = wot_ref[...]
    wofm = jnp.concatenate(
        [jnp.zeros((5, 32), jnp.float32), wot[:, 32:34],
         jnp.zeros((5, XW - 34), jnp.float32)], axis=1)
    outt = (_dotT(wot[:, :32], h) + _dotT(wofm, xm)
            + lax.broadcast_in_dim(bo_ref[...], (5, B), (0,)))
    out_ref[...] = outt


_tc_dnn = pl.pallas_call(
    _tc_body,
    out_shape=jax.ShapeDtypeStruct((5, B), jnp.float32),
)


def kernel(uid, gid, job, sex, age, pubtime, recent_films, category,
           fm1_uid, fm1_gid, fm1_job, fm1_sex, fm1_age, fm1_pub,
           user_emb, group_emb, cat_emb,
           fm2_job, fm2_sex, fm2_age, fm2_pub,
           W1, b1, g1, be1, W2, b2, g2, be2, W3, b3, g3, be3, Wo, bo):
    i32 = jnp.int32
    meta = jnp.concatenate(
        [recent_films.astype(i32), category.astype(i32),
         uid.astype(i32)[:, None], gid.astype(i32)[:, None],
         job.astype(i32)[:, None], sex.astype(i32)[:, None],
         age.astype(i32)[:, None], pubtime.astype(i32)[:, None],
         jnp.zeros((B, 2), i32)], axis=1).reshape(B * MW // 128, 128)
    tables = dict(fm1_uid=fm1_uid, fm1_gid=fm1_gid, fm1_job=fm1_job,
                  fm1_sex=fm1_sex, fm1_age=fm1_age, fm1_pub=fm1_pub,
                  user=user_emb, group=group_emb, cat=cat_emb,
                  f2job=fm2_job, f2sex=fm2_sex, f2age=fm2_age,
                  f2pub=fm2_pub)
    pieces = [tables[name].reshape(-1) for name in _TABLE_ARG_ORDER]
    pieces.append(jnp.zeros((TAB_LEN - _cur,), jnp.float32))
    tab = jnp.concatenate(pieces)
    xx = _sc_gather(meta, tab)
    outt = _tc_dnn(xx, W1, b1, g1, be1, W2, b2, g2, be2,
                   W3.T, b3, g3, be3, Wo.T, bo)
    return jnp.transpose(outt)


# speedup vs baseline: 1.0465x; 1.0465x over previous
import functools

import jax
import jax.numpy as jnp
from jax import lax
from jax.experimental import pallas as pl
from jax.experimental.pallas import tpu as pltpu
from jax.experimental.pallas import tpu_sc as plsc

B = 4096
HIST = 20
NCAT = 4
EMB = 4
XW = 128
MW = 32

NC, NS, L = 2, 16, 16
NW = NC * NS
BPW = B // NW
NCHUNK = BPW // L

_REC0 = 0
_CAT0 = HIST
_UID, _GID, _JOB, _SEX, _AGE, _PUB = 24, 25, 26, 27, 28, 29

_OFF = {}
_cur = 0
for _name, _len in [
    ("fm1_uid", 944), ("fm1_gid", 1683), ("fm1_job", 22), ("fm1_sex", 2),
    ("fm1_age", 5), ("fm1_pub", 5),
    ("user", 944 * EMB), ("group", 1683 * EMB), ("cat", 20 * EMB),
    ("f2job", 22 * EMB), ("f2sex", 2 * EMB), ("f2age", 5 * EMB),
    ("f2pub", 5 * EMB),
]:
    _OFF[_name] = _cur
    _cur += _len
TAB_LEN = (_cur + 7) // 8 * 8

_TABLE_ARG_ORDER = ["fm1_uid", "fm1_gid", "fm1_job", "fm1_sex", "fm1_age",
                    "fm1_pub", "user", "group", "cat", "f2job", "f2sex",
                    "f2age", "f2pub"]


def _sc_body(meta_h, tab_h, xx_out, meta_v, tab_v, x_v, sem):
    wid = lax.axis_index("s") * NC + lax.axis_index("c")
    base = wid * BPW

    mrow = pl.multiple_of(base * MW // 128, 8)
    cps = [pltpu.make_async_copy(meta_h.at[pl.ds(mrow, BPW * MW // 128), :],
                                 meta_v, sem),
           pltpu.make_async_copy(tab_h, tab_v, sem)]
    for c in cps:
        c.start()
    for c in cps:
        c.wait()

    iota = jnp.arange(L, dtype=jnp.int32)

    def put(xb, col, val):
        plsc.store_scatter(x_v, [xb + col], val)

    @plsc.parallel_loop(0, NCHUNK, unroll=2)
    def chunk(i):
        sidx = i * L + iota
        xb = lax.shift_left(sidx, 7)
        mr = lax.shift_right_logical(sidx, 2)
        mc = lax.shift_left(jnp.bitwise_and(sidx, 3), 5)

        def midx(c):
            return plsc.load_gather(meta_v, [mr, mc + c])

        uid = midx(_UID)
        gid = midx(_GID)
        job = midx(_JOB)
        sex = midx(_SEX)
        age = midx(_AGE)
        pub = midx(_PUB)

        def lg(idx):
            return plsc.load_gather(tab_v, [idx])

        fm1 = (lg(_OFF["fm1_uid"] + uid) + lg(_OFF["fm1_gid"] + gid)
               + lg(_OFF["fm1_job"] + job) + lg(_OFF["fm1_sex"] + sex)
               + lg(_OFF["fm1_age"] + age) + lg(_OFF["fm1_pub"] + pub))
        put(xb, 32, fm1)

        f_base = [
            _OFF["f2job"] + job * EMB,
            _OFF["f2sex"] + sex * EMB,
            _OFF["f2age"] + age * EMB,
            _OFF["f2pub"] + pub * EMB,
            _OFF["user"] + uid * EMB,
            _OFF["group"] + gid * EMB,
        ]
        rec_base = [_OFF["group"] + midx(_REC0 + h) * EMB
                    for h in range(HIST)]
        cat_base = [_OFF["cat"] + midx(_CAT0 + c) * EMB
                    for c in range(NCAT)]

        fm2_acc = jnp.zeros((L,), jnp.float32)
        for d in range(EMB):
            e = [lg(fb + d) for fb in f_base]
            hsum = lg(rec_base[0] + d)
            for rb in rec_base[1:]:
                hsum = hsum + lg(rb + d)
            csum = lg(cat_base[0] + d)
            for cb in cat_base[1:]:
                csum = csum + lg(cb + d)
            e.append(hsum)
            e.append(csum * 0.25)
            s = e[0]
            sq = e[0] * e[0]
            for ef in e[1:]:
                s = s + ef
                sq = sq + ef * ef
            fm2_acc = fm2_acc + (s * s - sq)
            for f in range(8):
                put(xb, f * EMB + d, e[f])
        put(xb, 33, fm2_acc * 0.5)

    pltpu.sync_copy(x_v, xx_out.at[pl.ds(base * XW, BPW * XW)])


_sc_gather = functools.partial(
    pl.kernel,
    out_type=jax.ShapeDtypeStruct((B * XW,), jnp.float32),
    mesh=plsc.VectorSubcoreMesh(core_axis_name="c", subcore_axis_name="s"),
    scratch_types=[
        pltpu.VMEM((BPW * MW // 128, 128), jnp.int32),
        pltpu.VMEM((TAB_LEN,), jnp.float32),
        pltpu.VMEM((BPW * XW,), jnp.float32),
        pltpu.SemaphoreType.DMA,
    ],
    compiler_params=pltpu.CompilerParams(needs_layout_passes=False),
)(_sc_body)


def _dotT(a, b):
    return lax.dot_general(a, b, (((1,), (1,)), ((), ())),
                           preferred_element_type=jnp.float32)


def _tc_body(xx_ref, w1_ref, b1_ref, g1_ref, be1_ref,
             w2_ref, b2_ref, g2_ref, be2_ref, w3t_ref, b3_ref, g3_ref,
             be3_ref, wot_ref, bo_ref, out_ref):
    ones = jnp.full((1, B), 1.0 / B, jnp.float32)

    def bn_relu(h, b, g, be):
        n = h.shape[1]
        h = h + b.reshape(1, n)
        m = jnp.dot(ones, h, preferred_element_type=jnp.float32)
        ms = jnp.dot(ones, h * h, preferred_element_type=jnp.float32)
        v = ms - m * m
        scale = g.reshape(1, n) / jnp.sqrt(v + 1e-5)
        shift = be.reshape(1, n) - m * scale
        return jnp.maximum(h * scale + shift, 0.0)

    xx = jnp.reshape(xx_ref[...], (B, XW))
    lane = lax.broadcasted_iota(jnp.int32, (B, XW), 1)
    xm = jnp.where(lane < 34, xx, 0.0)
    w1p = jnp.concatenate(
        [w1_ref[...], jnp.zeros((XW - 32, 256), jnp.float32)], axis=0)
    h = bn_relu(jnp.dot(xm, w1p, preferred_element_type=jnp.float32),
                b1_ref[...], g1_ref[...], be1_ref[...])
    h = bn_relu(jnp.dot(h, w2_ref[...], preferred_element_type=jnp.float32),
                b2_ref[...], g2_ref[...], be2_ref[...])
    h = bn_relu(_dotT(h, w3t_ref[...]),
                b3_ref[...], g3_ref[...], be3_ref[...])
    wot = wot_ref[...]
    wofm = jnp.concatenate(
        [jnp.zeros((5, 32), jnp.float32), wot[:, 32:34],
         jnp.zeros((5, XW - 34), jnp.float32)], axis=1)
    outt = (_dotT(wot[:, :32], h) + _dotT(wofm, xm)
            + lax.broadcast_in_dim(bo_ref[...], (5, B), (0,)))
    out_ref[...] = outt


_tc_dnn = pl.pallas_call(
    _tc_body,
    out_shape=jax.ShapeDtypeStruct((5, B), jnp.float32),
)


def kernel(uid, gid, job, sex, age, pubtime, recent_films, category,
           fm1_uid, fm1_gid, fm1_job, fm1_sex, fm1_age, fm1_pub,
           user_emb, group_emb, cat_emb,
           fm2_job, fm2_sex, fm2_age, fm2_pub,
           W1, b1, g1, be1, W2, b2, g2, be2, W3, b3, g3, be3, Wo, bo):
    i32 = jnp.int32
    meta = jnp.concatenate(
        [recent_films.astype(i32), category.astype(i32),
         uid.astype(i32)[:, None], gid.astype(i32)[:, None],
         job.astype(i32)[:, None], sex.astype(i32)[:, None],
         age.astype(i32)[:, None], pubtime.astype(i32)[:, None],
         jnp.zeros((B, 2), i32)], axis=1).reshape(B * MW // 128, 128)
    tables = dict(fm1_uid=fm1_uid, fm1_gid=fm1_gid, fm1_job=fm1_job,
                  fm1_sex=fm1_sex, fm1_age=fm1_age, fm1_pub=fm1_pub,
                  user=user_emb, group=group_emb, cat=cat_emb,
                  f2job=fm2_job, f2sex=fm2_sex, f2age=fm2_age,
                  f2pub=fm2_pub)
    pieces = [tables[name].reshape(-1) for name in _TABLE_ARG_ORDER]
    pieces.append(jnp.zeros((TAB_LEN - _cur,), jnp.float32))
    tab = jnp.concatenate(pieces)
    xx = _sc_gather(meta, tab)
    outt = _tc_dnn(xx, W1, b1, g1, be1, W2, b2, g2, be2,
                   W3.T, b3, g3, be3, Wo.T, bo)
    return jnp.transpose(outt)
